# baseline (speedup 1.0000x reference)
import functools

import jax
import jax.numpy as jnp
from jax import lax
from jax.experimental import pallas as pl
from jax.experimental.pallas import tpu as pltpu
from jax.experimental.pallas import tpu_sc as plsc

N = 1_000_000
D = 32
Q = 8
NC = 2
NS = 16
NW = NC * NS
L = 16

CHUNK = 1024
TPC = 32
NCHUNKS = (N + CHUNK - 1) // CHUNK
TILE = 64
EPS = 1e-8
EPS2 = EPS * EPS
NEG_INF = float("-inf")


def _sc_body(keys, qsplat, out_t, out_i, buf0, buf1, qv, res_t, res_i,
             sem0, sem1):
    cid = lax.axis_index("c")
    sid = lax.axis_index("s")
    wid = cid * NS + sid

    pltpu.sync_copy(qsplat, qv)

    def row0(t):
        c = jnp.minimum(wid + NW * jnp.minimum(t, TPC - 2), NCHUNKS - 1)
        return jnp.minimum(c * CHUNK, N - CHUNK)

    iota = lax.iota(jnp.int32, L)

    def process_chunk(t, buf, carry):
        base = row0(t)

        def tile_body(tile, carry):
            best_t, best_i = carry
            rows = [tile * TILE + j * L + iota for j in range(TILE // L)]
            nj = len(rows)

            accs = [jnp.zeros((L,), jnp.float32) for _ in range(nj * (Q + 1))]
            for d in range(D):
                col = jnp.full((L,), d, jnp.int32)
                v = [plsc.load_gather(buf, [r, col]) for r in rows]
                for q in range(Q):
                    s = qv[pl.ds((q * D + d) * L, L)]
                    for j in range(nj):
                        accs[j * (Q + 1) + q] = accs[j * (Q + 1) + q] + v[j] * s
                for j in range(nj):
                    accs[j * (Q + 1) + Q] = accs[j * (Q + 1) + Q] + v[j] * v[j]

            best_t = list(best_t)
            best_i = list(best_i)
            for j in range(nj):
                rcp = 1.0 / jnp.maximum(accs[j * (Q + 1) + Q], EPS2)
                idx_vec = base + rows[j]
                for q in range(Q):
                    dot = accs[j * (Q + 1) + q]
                    tval = dot * jnp.abs(dot) * rcp
                    better = tval > best_t[q]
                    best_t[q] = jnp.where(better, tval, best_t[q])
                    best_i[q] = jnp.where(better, idx_vec, best_i[q])
            return (tuple(best_t), tuple(best_i))

        return lax.fori_loop(0, CHUNK // TILE, tile_body, carry)

    best_t = tuple(jnp.full((L,), NEG_INF, jnp.float32) for _ in range(Q))
    best_i = tuple(jnp.zeros((L,), jnp.int32) for _ in range(Q))
    carry = (best_t, best_i)

    bufs, sems = (buf0, buf1), (sem0, sem1)
    pltpu.async_copy(keys.at[pl.ds(row0(0), CHUNK)], buf0, sem0)

    def outer(i, carry):
        for b in range(2):
            t = 2 * i + b
            nxt = jnp.minimum(t + 1, TPC - 1)
            pltpu.async_copy(keys.at[pl.ds(row0(nxt), CHUNK)],
                             bufs[1 - b], sems[1 - b])
            pltpu.make_async_copy(keys.at[pl.ds(row0(t), CHUNK)],
                                  bufs[b], sems[b]).wait()
            carry = process_chunk(t, bufs[b], carry)
        return carry

    carry = lax.fori_loop(0, TPC // 2, outer, carry)
    pltpu.make_async_copy(keys.at[pl.ds(row0(TPC - 1), CHUNK)], bufs[0],
                          sems[0]).wait()

    best_t, best_i = carry
    for q in range(Q):
        res_t[q, :] = best_t[q]
        res_i[q, :] = best_i[q]
    pltpu.sync_copy(res_t, out_t.at[wid])
    pltpu.sync_copy(res_i, out_i.at[wid])


def _run_sc(keys, qsplat):
    mesh = plsc.VectorSubcoreMesh(core_axis_name="c", subcore_axis_name="s",
                                  num_cores=NC, num_subcores=NS)
    f = pl.kernel(
        _sc_body,
        out_type=(
            jax.ShapeDtypeStruct((NW, Q, L), jnp.float32),
            jax.ShapeDtypeStruct((NW, Q, L), jnp.int32),
        ),
        mesh=mesh,
        scratch_types=[
            pltpu.VMEM((CHUNK, D), jnp.float32),
            pltpu.VMEM((CHUNK, D), jnp.float32),
            pltpu.VMEM((Q * D * L,), jnp.float32),
            pltpu.VMEM((Q, L), jnp.float32),
            pltpu.VMEM((Q, L), jnp.int32),
            pltpu.SemaphoreType.DMA,
            pltpu.SemaphoreType.DMA,
        ],
        compiler_params=pltpu.CompilerParams(
            needs_layout_passes=False, use_tc_tiling_on_sc=False),
    )
    return f(keys, qsplat)


@jax.jit
def kernel(queries, keys):
    qn = queries / jnp.maximum(
        jnp.linalg.norm(queries, axis=-1, keepdims=True), EPS)
    qsplat = jnp.broadcast_to(qn.reshape(Q, D, 1), (Q, D, L)).reshape(-1)

    t_c, i_c = _run_sc(keys, qsplat)

    sims = jnp.sign(t_c) * jnp.sqrt(jnp.abs(t_c))
    sims = sims.transpose(1, 0, 2).reshape(Q, NW * L)
    idx = i_c.transpose(1, 0, 2).reshape(Q, NW * L)
    vals = jnp.max(sims, axis=1)
    at_max = sims == vals[:, None]
    best_idx = jnp.min(jnp.where(at_max, idx, N), axis=1)
    return vals, best_idx.astype(jnp.int32)


# speedup vs baseline: 1.2036x; 1.2036x over previous
import functools

import jax
import jax.numpy as jnp
from jax import lax
from jax.experimental import pallas as pl
from jax.experimental.pallas import tpu as pltpu
from jax.experimental.pallas import tpu_sc as plsc

N = 1_000_000
D = 32
Q = 8
NC = 2
NS = 16
NW = NC * NS
L = 16

CHUNK = 1024
TPC = 18
N_SC = NW * TPC * CHUNK
N_TC = N - N_SC
BK = 16384
G_TC = (N_TC + BK - 1) // BK
TILE = 64
EPS = 1e-8
EPS2 = EPS * EPS
NEG_INF = float("-inf")


def _tc_body(qn_ref, keys_ref, out_v, out_i):
    pi = pl.program_id(0)

    k = keys_ref[...]
    d = lax.dot_general(qn_ref[...], k, (((1,), (1,)), ((), ())),
                        precision=lax.Precision.HIGHEST,
                        preferred_element_type=jnp.float32)
    ksq = k * k
    ones = jnp.ones((Q, D), jnp.float32)
    s8 = lax.dot_general(ones, ksq, (((1,), (1,)), ((), ())),
                         precision=lax.Precision.HIGHEST,
                         preferred_element_type=jnp.float32)
    rs = lax.rsqrt(jnp.maximum(s8[0:1], EPS2))
    sims = d * rs

    row = pi * BK + lax.broadcasted_iota(jnp.int32, (Q, BK), 1)
    sims = jnp.where(row < N_TC, sims, NEG_INF)

    m = jnp.max(sims, axis=1, keepdims=True)
    cand = jnp.where(sims == m, row, N)
    out_v[...] = m[None]
    out_i[...] = jnp.min(cand, axis=1, keepdims=True)[None]


def _run_tc(qn, keys):
    return pl.pallas_call(
        _tc_body,
        grid=(G_TC,),
        in_specs=[
            pl.BlockSpec((Q, D), lambda i: (0, 0)),
            pl.BlockSpec((BK, D), lambda i: (i, 0)),
        ],
        out_specs=[
            pl.BlockSpec((1, Q, 1), lambda i: (i, 0, 0)),
            pl.BlockSpec((1, Q, 1), lambda i: (i, 0, 0)),
        ],
        out_shape=[
            jax.ShapeDtypeStruct((G_TC, Q, 1), jnp.float32),
            jax.ShapeDtypeStruct((G_TC, Q, 1), jnp.int32),
        ],
    )(qn, keys)


def _sc_body(keys, qsplat, out_t, out_i, buf0, buf1, qv, res_t, res_i,
             sem0, sem1):
    cid = lax.axis_index("c")
    sid = lax.axis_index("s")
    wid = cid * NS + sid

    pltpu.sync_copy(qsplat, qv)

    def row0(t):
        return N_TC + (wid + NW * t) * CHUNK

    iota = lax.iota(jnp.int32, L)

    def process_chunk(t, buf, carry):
        base = row0(t)

        def tile_body(tile, carry):
            best_t, best_i = carry
            rows = [tile * TILE + j * L + iota for j in range(TILE // L)]
            nj = len(rows)

            accs = [jnp.zeros((L,), jnp.float32) for _ in range(nj * (Q + 1))]
            for d in range(D):
                col = jnp.full((L,), d, jnp.int32)
                v = [plsc.load_gather(buf, [r, col]) for r in rows]
                for q in range(Q):
                    s = qv[pl.ds((q * D + d) * L, L)]
                    for j in range(nj):
                        accs[j * (Q + 1) + q] = accs[j * (Q + 1) + q] + v[j] * s
                for j in range(nj):
                    accs[j * (Q + 1) + Q] = accs[j * (Q + 1) + Q] + v[j] * v[j]

            best_t = list(best_t)
            best_i = list(best_i)
            for j in range(nj):
                rcp = 1.0 / jnp.maximum(accs[j * (Q + 1) + Q], EPS2)
                idx_vec = base + rows[j]
                for q in range(Q):
                    dot = accs[j * (Q + 1) + q]
                    tval = dot * jnp.abs(dot) * rcp
                    better = tval > best_t[q]
                    best_t[q] = jnp.where(better, tval, best_t[q])
                    best_i[q] = jnp.where(better, idx_vec, best_i[q])
            return (tuple(best_t), tuple(best_i))

        return lax.fori_loop(0, CHUNK // TILE, tile_body, carry)

    best_t = tuple(jnp.full((L,), NEG_INF, jnp.float32) for _ in range(Q))
    best_i = tuple(jnp.zeros((L,), jnp.int32) for _ in range(Q))
    carry = (best_t, best_i)

    bufs, sems = (buf0, buf1), (sem0, sem1)
    pltpu.async_copy(keys.at[pl.ds(row0(0), CHUNK)], buf0, sem0)

    def outer(i, carry):
        for b in range(2):
            t = 2 * i + b
            nxt = jnp.minimum(t + 1, TPC - 1)
            pltpu.async_copy(keys.at[pl.ds(row0(nxt), CHUNK)],
                             bufs[1 - b], sems[1 - b])
            pltpu.make_async_copy(keys.at[pl.ds(row0(t), CHUNK)],
                                  bufs[b], sems[b]).wait()
            carry = process_chunk(t, bufs[b], carry)
        return carry

    carry = lax.fori_loop(0, TPC // 2, outer, carry)
    pltpu.make_async_copy(keys.at[pl.ds(row0(TPC - 1), CHUNK)], bufs[0],
                          sems[0]).wait()

    best_t, best_i = carry
    for q in range(Q):
        res_t[q, :] = best_t[q]
        res_i[q, :] = best_i[q]
    pltpu.sync_copy(res_t, out_t.at[wid])
    pltpu.sync_copy(res_i, out_i.at[wid])


def _run_sc(keys, qsplat):
    mesh = plsc.VectorSubcoreMesh(core_axis_name="c", subcore_axis_name="s",
                                  num_cores=NC, num_subcores=NS)
    f = pl.kernel(
        _sc_body,
        out_type=(
            jax.ShapeDtypeStruct((NW, Q, L), jnp.float32),
            jax.ShapeDtypeStruct((NW, Q, L), jnp.int32),
        ),
        mesh=mesh,
        scratch_types=[
            pltpu.VMEM((CHUNK, D), jnp.float32),
            pltpu.VMEM((CHUNK, D), jnp.float32),
            pltpu.VMEM((Q * D * L,), jnp.float32),
            pltpu.VMEM((Q, L), jnp.float32),
            pltpu.VMEM((Q, L), jnp.int32),
            pltpu.SemaphoreType.DMA,
            pltpu.SemaphoreType.DMA,
        ],
        compiler_params=pltpu.CompilerParams(
            needs_layout_passes=False, use_tc_tiling_on_sc=False),
    )
    return f(keys, qsplat)


@jax.jit
def kernel(queries, keys):
    qn = queries / jnp.maximum(
        jnp.linalg.norm(queries, axis=-1, keepdims=True), EPS)
    qsplat = jnp.broadcast_to(qn.reshape(Q, D, 1), (Q, D, L)).reshape(-1)

    t_c, i_c = _run_sc(keys, qsplat)
    tc_v, tc_i = _run_tc(qn, keys)

    sc_sims = jnp.sign(t_c) * jnp.sqrt(jnp.abs(t_c))
    sc_sims = sc_sims.transpose(1, 0, 2).reshape(Q, NW * L)
    sc_idx = i_c.transpose(1, 0, 2).reshape(Q, NW * L)
    tc_v = tc_v.reshape(G_TC, Q).T
    tc_i = tc_i.reshape(G_TC, Q).T
    sims = jnp.concatenate([sc_sims, tc_v], axis=1)
    idx = jnp.concatenate([sc_idx, tc_i], axis=1)
    vals = jnp.max(sims, axis=1)
    at_max = sims == vals[:, None]
    best_idx = jnp.min(jnp.where(at_max, idx, N), axis=1)
    return vals, best_idx.astype(jnp.int32)


# speedup vs baseline: 1.2282x; 1.0204x over previous
import functools

import jax
import jax.numpy as jnp
from jax import lax
from jax.experimental import pallas as pl
from jax.experimental.pallas import tpu as pltpu
from jax.experimental.pallas import tpu_sc as plsc

N = 1_000_000
D = 32
Q = 8
NC = 2
NS = 16
NW = NC * NS
L = 16

CHUNK = 1024
TPC = 18
N_SC = NW * TPC * CHUNK
N_TC = N - N_SC
BK = 8192
G_TC = (N_TC + BK - 1) // BK
TILE = 64
EPS = 1e-8
EPS2 = EPS * EPS
NEG_INF = float("-inf")


def _tc_body(qn_ref, keys_ref, out_v, out_i):
    pi = pl.program_id(0)

    k = keys_ref[...]

    def split(a):
        ai = lax.bitcast_convert_type(a, jnp.int32)
        hi = lax.bitcast_convert_type(ai & jnp.int32(-65536), jnp.float32)
        return hi.astype(jnp.bfloat16), (a - hi).astype(jnp.bfloat16)

    def dot3(a, b):
        ah, al = split(a)
        bh, bl = split(b)
        dims = (((1,), (1,)), ((), ()))
        f32 = jnp.float32
        return (lax.dot_general(ah, bh, dims, preferred_element_type=f32)
                + lax.dot_general(ah, bl, dims, preferred_element_type=f32)
                + lax.dot_general(al, bh, dims, preferred_element_type=f32))

    d = dot3(qn_ref[...], k)
    ksq = k * k
    ones = jnp.ones((Q, D), jnp.float32)
    s8 = dot3(ones, ksq)
    rs = lax.rsqrt(jnp.maximum(s8[0:1], EPS2))
    sims = d * rs

    row = pi * BK + lax.broadcasted_iota(jnp.int32, (Q, BK), 1)
    sims = jnp.where(row < N_TC, sims, NEG_INF)

    m = jnp.max(sims, axis=1, keepdims=True)
    cand = jnp.where(sims == m, row, N)
    out_v[...] = m[None]
    out_i[...] = jnp.min(cand, axis=1, keepdims=True)[None]


def _run_tc(qn, keys):
    return pl.pallas_call(
        _tc_body,
        grid=(G_TC,),
        in_specs=[
            pl.BlockSpec((Q, D), lambda i: (0, 0)),
            pl.BlockSpec((BK, D), lambda i: (i, 0)),
        ],
        out_specs=[
            pl.BlockSpec((1, Q, 1), lambda i: (i, 0, 0)),
            pl.BlockSpec((1, Q, 1), lambda i: (i, 0, 0)),
        ],
        out_shape=[
            jax.ShapeDtypeStruct((G_TC, Q, 1), jnp.float32),
            jax.ShapeDtypeStruct((G_TC, Q, 1), jnp.int32),
        ],
    )(qn, keys)


def _sc_body(keys, qsplat, out_t, out_i, buf0, buf1, qv, res_t, res_i,
             sem0, sem1):
    cid = lax.axis_index("c")
    sid = lax.axis_index("s")
    wid = cid * NS + sid

    pltpu.sync_copy(qsplat, qv)

    def row0(t):
        return N_TC + (wid + NW * t) * CHUNK

    iota = lax.iota(jnp.int32, L)

    def process_chunk(t, buf, carry):
        base = row0(t)

        def tile_body(tile, carry):
            best_t, best_i = carry
            rows = [tile * TILE + j * L + iota for j in range(TILE // L)]
            nj = len(rows)

            accs = [jnp.zeros((L,), jnp.float32) for _ in range(nj * (Q + 1))]
            for d in range(D):
                col = jnp.full((L,), d, jnp.int32)
                v = [plsc.load_gather(buf, [r, col]) for r in rows]
                for q in range(Q):
                    s = qv[pl.ds((q * D + d) * L, L)]
                    for j in range(nj):
                        accs[j * (Q + 1) + q] = accs[j * (Q + 1) + q] + v[j] * s
                for j in range(nj):
                    accs[j * (Q + 1) + Q] = accs[j * (Q + 1) + Q] + v[j] * v[j]

            best_t = list(best_t)
            best_i = list(best_i)
            for j in range(nj):
                rcp = 1.0 / jnp.maximum(accs[j * (Q + 1) + Q], EPS2)
                idx_vec = base + rows[j]
                for q in range(Q):
                    dot = accs[j * (Q + 1) + q]
                    tval = dot * jnp.abs(dot) * rcp
                    better = tval > best_t[q]
                    best_t[q] = jnp.where(better, tval, best_t[q])
                    best_i[q] = jnp.where(better, idx_vec, best_i[q])
            return (tuple(best_t), tuple(best_i))

        return lax.fori_loop(0, CHUNK // TILE, tile_body, carry)

    best_t = tuple(jnp.full((L,), NEG_INF, jnp.float32) for _ in range(Q))
    best_i = tuple(jnp.zeros((L,), jnp.int32) for _ in range(Q))
    carry = (best_t, best_i)

    bufs, sems = (buf0, buf1), (sem0, sem1)
    pltpu.async_copy(keys.at[pl.ds(row0(0), CHUNK)], buf0, sem0)

    def outer(i, carry):
        for b in range(2):
            t = 2 * i + b
            nxt = jnp.minimum(t + 1, TPC - 1)
            pltpu.async_copy(keys.at[pl.ds(row0(nxt), CHUNK)],
                             bufs[1 - b], sems[1 - b])
            pltpu.make_async_copy(keys.at[pl.ds(row0(t), CHUNK)],
                                  bufs[b], sems[b]).wait()
            carry = process_chunk(t, bufs[b], carry)
        return carry

    carry = lax.fori_loop(0, TPC // 2, outer, carry)
    pltpu.make_async_copy(keys.at[pl.ds(row0(TPC - 1), CHUNK)], bufs[0],
                          sems[0]).wait()

    best_t, best_i = carry
    for q in range(Q):
        res_t[q, :] = best_t[q]
        res_i[q, :] = best_i[q]
    pltpu.sync_copy(res_t, out_t.at[wid])
    pltpu.sync_copy(res_i, out_i.at[wid])


def _run_sc(keys, qsplat):
    mesh = plsc.VectorSubcoreMesh(core_axis_name="c", subcore_axis_name="s",
                                  num_cores=NC, num_subcores=NS)
    f = pl.kernel(
        _sc_body,
        out_type=(
            jax.ShapeDtypeStruct((NW, Q, L), jnp.float32),
            jax.ShapeDtypeStruct((NW, Q, L), jnp.int32),
        ),
        mesh=mesh,
        scratch_types=[
            pltpu.VMEM((CHUNK, D), jnp.float32),
            pltpu.VMEM((CHUNK, D), jnp.float32),
            pltpu.VMEM((Q * D * L,), jnp.float32),
            pltpu.VMEM((Q, L), jnp.float32),
            pltpu.VMEM((Q, L), jnp.int32),
            pltpu.SemaphoreType.DMA,
            pltpu.SemaphoreType.DMA,
        ],
        compiler_params=pltpu.CompilerParams(
            needs_layout_passes=False, use_tc_tiling_on_sc=False),
    )
    return f(keys, qsplat)


@jax.jit
def kernel(queries, keys):
    qn = queries / jnp.maximum(
        jnp.linalg.norm(queries, axis=-1, keepdims=True), EPS)
    qsplat = jnp.broadcast_to(qn.reshape(Q, D, 1), (Q, D, L)).reshape(-1)

    t_c, i_c = _run_sc(keys, qsplat)
    tc_v, tc_i = _run_tc(qn, keys)

    sc_sims = jnp.sign(t_c) * jnp.sqrt(jnp.abs(t_c))
    sc_sims = sc_sims.transpose(1, 0, 2).reshape(Q, NW * L)
    sc_idx = i_c.transpose(1, 0, 2).reshape(Q, NW * L)
    tc_v = tc_v.reshape(G_TC, Q).T
    tc_i = tc_i.reshape(G_TC, Q).T
    sims = jnp.concatenate([sc_sims, tc_v], axis=1)
    idx = jnp.concatenate([sc_idx, tc_i], axis=1)
    vals = jnp.max(sims, axis=1)
    at_max = sims == vals[:, None]
    best_idx = jnp.min(jnp.where(at_max, idx, N), axis=1)
    return vals, best_idx.astype(jnp.int32)
